# Initial kernel scaffold; baseline (speedup 1.0000x reference)
import jax
import jax.numpy as jnp
from jax.experimental import pallas as pl


def kernel(x, token_emb, pos_emb):
    raise NotImplementedError("write your pallas kernel here")


# speedup vs baseline: 1.4882x; 1.4882x over previous
import functools

import jax
import jax.numpy as jnp
from jax import lax
from jax.experimental import pallas as pl
from jax.experimental.pallas import tpu as pltpu
from jax.experimental.pallas import tpu_sc as plsc

EMBED = 32
T = 200

NW = 32
CHUNK = 800
SUB = 100
NSUB = CHUNK // SUB
PERIODS = CHUNK // T


def _sc_embed(n_rows):
    n_chunks = n_rows // (NW * CHUNK)
    mesh = plsc.VectorSubcoreMesh(core_axis_name="c", subcore_axis_name="s")

    @functools.partial(
        pl.kernel,
        mesh=mesh,
        compiler_params=pltpu.CompilerParams(use_tc_tiling_on_sc=False),
        out_type=jax.ShapeDtypeStruct((NW, n_chunks, CHUNK, EMBED), jnp.float32),
        scratch_types=[
            pltpu.VMEM((n_chunks, NSUB, SUB), jnp.int32),
            pltpu.VMEM((2, CHUNK, EMBED), jnp.float32),
            pltpu.VMEM((T, EMBED), jnp.float32),
            pltpu.SemaphoreType.DMA,
            pltpu.SemaphoreType.DMA,
            pltpu.SemaphoreType.DMA,
            pltpu.SemaphoreType.DMA,
        ],
    )
    def k(table_hbm, idx_hbm, pos_hbm, out_hbm,
          idx_v, rows_v, pos_v, gsem0, gsem1, osem0, osem1):
        wid = lax.axis_index("s") * 2 + lax.axis_index("c")
        gsem = (gsem0, gsem1)
        osem = (osem0, osem1)

        def fire_gathers(b, g):
            for j in range(NSUB):
                pltpu.async_copy(
                    table_hbm.at[idx_v.at[g, j]],
                    rows_v.at[b, pl.ds(j * SUB, SUB)],
                    gsem[b],
                )

        def drain_gathers(b):
            pltpu.make_async_copy(out_hbm.at[wid, 0], rows_v.at[b], gsem[b]).wait()

        def fire_out(b, g):
            pltpu.async_copy(rows_v.at[b], out_hbm.at[wid, g], osem[b])

        def drain_out(b):
            pltpu.make_async_copy(rows_v.at[b], out_hbm.at[wid, 0], osem[b]).wait()

        def pos_add(b):
            def row_body(r, c):
                p0 = pos_v[r, pl.ds(0, 16)]
                p1 = pos_v[r, pl.ds(16, 16)]
                for o in range(PERIODS):
                    rr = o * T + r
                    rows_v[b, rr, pl.ds(0, 16)] = rows_v[b, rr, pl.ds(0, 16)] + p0
                    rows_v[b, rr, pl.ds(16, 16)] = rows_v[b, rr, pl.ds(16, 16)] + p1
                return c

            lax.fori_loop(0, T, row_body, 0)

        pltpu.sync_copy(pos_hbm, pos_v)
        pltpu.sync_copy(idx_hbm.at[wid], idx_v)
        fire_gathers(0, 0)

        def outer(i, carry):
            for b in range(2):
                g = 2 * i + b
                pb = 1 - b

                @pl.when(g + 1 < n_chunks)
                def _stage():
                    @pl.when(g >= 1)
                    def _free():
                        drain_out(pb)

                    fire_gathers(pb, g + 1)

                drain_gathers(b)
                pos_add(b)
                fire_out(b, g)
            return carry

        lax.fori_loop(0, n_chunks // 2, outer, 0)
        drain_out(0)
        drain_out(1)

    return k


def kernel(x, token_emb, pos_emb):
    B, T_ = x.shape
    n = B * T_
    idx = x.reshape(-1).astype(jnp.int32).reshape(NW, n // (NW * CHUNK), NSUB, SUB)
    out = _sc_embed(n)(token_emb, idx, pos_emb)
    return out.reshape(B, T_, EMBED)


# speedup vs baseline: 1.4905x; 1.0016x over previous
import functools

import jax
import jax.numpy as jnp
from jax import lax
from jax.experimental import pallas as pl
from jax.experimental.pallas import tpu as pltpu
from jax.experimental.pallas import tpu_sc as plsc

EMBED = 32
T = 200
B = 4096

NW = 32
BPW = B // NW
BPC = 4
NCH = BPW // BPC
SUBS = ((0, 128), (128, 72))


def _sc_embed():
    mesh = plsc.VectorSubcoreMesh(core_axis_name="c", subcore_axis_name="s")

    @functools.partial(
        pl.kernel,
        mesh=mesh,
        compiler_params=pltpu.CompilerParams(use_tc_tiling_on_sc=False),
        out_type=jax.ShapeDtypeStruct((B, T, EMBED), jnp.float32),
        scratch_types=[
            pltpu.VMEM((BPW, T), jnp.int32),
            pltpu.VMEM((2, BPC, T, EMBED), jnp.float32),
            pltpu.VMEM((T, EMBED), jnp.float32),
            pltpu.SemaphoreType.DMA,
            pltpu.SemaphoreType.DMA,
            pltpu.SemaphoreType.DMA,
            pltpu.SemaphoreType.DMA,
        ],
    )
    def k(table_hbm, x_hbm, pos_hbm, out_hbm,
          idx_v, rows_v, pos_v, gsem0, gsem1, osem0, osem1):
        wid = lax.axis_index("s") * 2 + lax.axis_index("c")
        b0 = wid * BPW
        gsem = (gsem0, gsem1)
        osem = (osem0, osem1)

        def fire_gathers(b, g):
            for q in range(BPC):
                for off, width in SUBS:
                    pltpu.async_copy(
                        table_hbm.at[idx_v.at[BPC * g + q, pl.ds(off, width)]],
                        rows_v.at[b, q, pl.ds(off, width)],
                        gsem[b],
                    )

        def drain_gathers(b):
            pltpu.make_async_copy(
                out_hbm.at[pl.ds(b0, BPC)], rows_v.at[b], gsem[b]).wait()

        def fire_out(b, g):
            pltpu.async_copy(
                rows_v.at[b], out_hbm.at[pl.ds(b0 + BPC * g, BPC)], osem[b])

        def drain_out(b):
            pltpu.make_async_copy(
                rows_v.at[b], out_hbm.at[pl.ds(b0, BPC)], osem[b]).wait()

        def pos_add(b):
            def row_body(r, c):
                p0 = pos_v[r, pl.ds(0, 16)]
                p1 = pos_v[r, pl.ds(16, 16)]
                for q in range(BPC):
                    rows_v[b, q, r, pl.ds(0, 16)] = (
                        rows_v[b, q, r, pl.ds(0, 16)] + p0)
                    rows_v[b, q, r, pl.ds(16, 16)] = (
                        rows_v[b, q, r, pl.ds(16, 16)] + p1)
                return c

            lax.fori_loop(0, T, row_body, 0)

        pltpu.sync_copy(pos_hbm, pos_v)
        pltpu.sync_copy(x_hbm.at[pl.ds(b0, BPW)], idx_v)
        fire_gathers(0, 0)

        def outer(i, carry):
            for b in range(2):
                g = 2 * i + b
                pb = 1 - b

                @pl.when(g + 1 < NCH)
                def _stage():
                    @pl.when(g >= 1)
                    def _free():
                        drain_out(pb)

                    fire_gathers(pb, g + 1)

                drain_gathers(b)
                pos_add(b)
                fire_out(b, g)
            return carry

        lax.fori_loop(0, NCH // 2, outer, 0)
        drain_out(0)
        drain_out(1)

    return k


def kernel(x, token_emb, pos_emb):
    return _sc_embed()(token_emb, x.astype(jnp.int32), pos_emb)
